# baseline (speedup 1.0000x reference)
import functools

import jax
import jax.numpy as jnp
from jax import lax
from jax.experimental import pallas as pl
from jax.experimental.pallas import tpu as pltpu
from jax.experimental.pallas import tpu_sc as plsc

B = 16384
D = 32
NC = 2
NS = 16
NW = NC * NS
BPW = B // NW


def _gather_body(uid, iid, cid, tu, ti, tc, out,
                 idx0, idx1, idx2, rows0, rows1, rows2, sem):
    wid = lax.axis_index("s") * NC + lax.axis_index("c")
    base = wid * BPW
    feats = ((uid, tu, idx0, rows0), (iid, ti, idx1, rows1),
             (cid, tc, idx2, rows2))
    for ids, _, idx_v, _ in feats:
        pltpu.sync_copy(ids.at[pl.ds(base, BPW)], idx_v)
    copies = [
        pltpu.async_copy(tab.at[idx_v], rows_v, sem)
        for _, tab, idx_v, rows_v in feats
    ]
    for f, cp in enumerate(copies):
        cp.wait()
        rows_v = feats[f][3]
        pltpu.sync_copy(rows_v, out.at[pl.ds(base, BPW), pl.ds(f * D, D)])


@jax.jit
def kernel(user_id, item_id, category_id, table_user_id, table_item_id,
           table_category_id):
    mesh = plsc.VectorSubcoreMesh(core_axis_name="c", subcore_axis_name="s")
    k = pl.kernel(
        _gather_body,
        out_type=jax.ShapeDtypeStruct((B, 3 * D), jnp.float32),
        mesh=mesh,
        scratch_types=[
            pltpu.VMEM((BPW,), jnp.int32),
            pltpu.VMEM((BPW,), jnp.int32),
            pltpu.VMEM((BPW,), jnp.int32),
            pltpu.VMEM((BPW, D), jnp.float32),
            pltpu.VMEM((BPW, D), jnp.float32),
            pltpu.VMEM((BPW, D), jnp.float32),
            pltpu.SemaphoreType.DMA,
        ],
        compiler_params=pltpu.CompilerParams(use_tc_tiling_on_sc=False),
    )
    return k(user_id + 1, item_id + 1, category_id + 1,
             table_user_id, table_item_id, table_category_id)


# speedup vs baseline: 1.1622x; 1.1622x over previous
import functools

import jax
import jax.numpy as jnp
from jax import lax
from jax.experimental import pallas as pl
from jax.experimental.pallas import tpu as pltpu
from jax.experimental.pallas import tpu_sc as plsc

B = 16384
D = 32
NC = 2
NS = 16
NW = NC * NS
BPW = B // NW
NBUF = 16


def _gather_body(uid, iid, cid, tu, ti, tc, out,
                 idx0, idx1, idx2, rows_v, grp, sem):
    wid = lax.axis_index("s") * NC + lax.axis_index("c")
    base = wid * BPW
    feats = ((uid, tu, idx0), (iid, ti, idx1), (cid, tc, idx2))
    for ids, _, idx_v in feats:
        pltpu.sync_copy(ids.at[pl.ds(base, BPW)], idx_v)
    for f, (_, tab, idx_v) in enumerate(feats):

        def outer(o, carry, tab=tab, idx_v=idx_v):
            v16 = idx_v[pl.ds(o * NBUF, NBUF)]
            for b in range(NBUF):
                v = v16[b]
                g8 = pl.multiple_of((v >> 3) * 8, 8)
                pltpu.async_copy(tab.at[pl.ds(g8, 8)], grp.at[b], sem)
            for b in range(NBUF):
                pltpu.make_async_copy(tab.at[pl.ds(0, 8)], grp.at[b],
                                      sem).wait()
            for b in range(NBUF):
                m = v16[b] & 7
                p = o * NBUF + b
                rows_v[p, pl.ds(0, 16)] = grp[b, m, pl.ds(0, 16)]
                rows_v[p, pl.ds(16, 16)] = grp[b, m, pl.ds(16, 16)]
            return carry

        lax.fori_loop(0, BPW // NBUF, outer, 0)
        pltpu.sync_copy(rows_v, out.at[f, pl.ds(base, BPW)])


@jax.jit
def kernel(user_id, item_id, category_id, table_user_id, table_item_id,
           table_category_id):
    mesh = plsc.VectorSubcoreMesh(core_axis_name="c", subcore_axis_name="s")
    k = pl.kernel(
        _gather_body,
        out_type=jax.ShapeDtypeStruct((3, B, D), jnp.float32),
        mesh=mesh,
        scratch_types=[
            pltpu.VMEM((BPW,), jnp.int32),
            pltpu.VMEM((BPW,), jnp.int32),
            pltpu.VMEM((BPW,), jnp.int32),
            pltpu.VMEM((BPW, D), jnp.float32),
            pltpu.VMEM((NBUF, 8, D), jnp.float32),
            pltpu.SemaphoreType.DMA,
        ],
    )
    out3 = k(user_id + 1, item_id + 1, category_id + 1,
             table_user_id, table_item_id, table_category_id)
    return jnp.concatenate([out3[0], out3[1], out3[2]], axis=1)
